# Initial kernel scaffold; baseline (speedup 1.0000x reference)
import jax
import jax.numpy as jnp
from jax.experimental import pallas as pl


def kernel(x, edge_index, W_self, b_self, W_neigh, b_neigh):
    raise NotImplementedError("write your pallas kernel here")


# speedup vs baseline: 6.9655x; 6.9655x over previous
import functools

import jax
import jax.numpy as jnp
from jax import lax
from jax.experimental import pallas as pl
from jax.experimental.pallas import tpu as pltpu
from jax.experimental.pallas import tpu_sc as plsc

N_NODES = 10000
N_EDGES = 320000
DIM = 128

NUM_CORES = 2
NUM_SUBCORES = 16
NW = NUM_CORES * NUM_SUBCORES

CHUNK = 128
N_CHUNKS = N_EDGES // CHUNK
MAX_CHUNKS_PER_W = -(-N_CHUNKS // NW)
ROWS_PER_TILE = 624
ROWS_REM_OFF = ROWS_PER_TILE * NUM_SUBCORES
ROWS_REM = N_NODES - ROWS_REM_OFF


def _sc_segment_sum(x, src, dst, zeros2d, zeros1d):
  mesh = plsc.VectorSubcoreMesh(
      core_axis_name="c", subcore_axis_name="s",
      num_cores=NUM_CORES, num_subcores=NUM_SUBCORES)

  @functools.partial(
      pl.kernel,
      out_type=[
          jax.ShapeDtypeStruct((NUM_CORES, N_NODES, DIM), jnp.float32),
          jax.ShapeDtypeStruct((NUM_CORES, N_NODES), jnp.float32),
      ],
      mesh=mesh,
      scratch_types=[
          pltpu.VMEM((CHUNK,), jnp.int32),
          pltpu.VMEM((CHUNK,), jnp.int32),
          pltpu.VMEM((CHUNK, DIM), jnp.float32),
          pltpu.VMEM((CHUNK,), jnp.float32),
          pltpu.VMEM_SHARED((N_NODES, DIM), jnp.float32),
          pltpu.VMEM_SHARED((N_NODES,), jnp.float32),
          pltpu.SemaphoreType.DMA,
      ],
  )
  def k(x_hbm, src_hbm, dst_hbm, z2_hbm, z1_hbm, parts_hbm, degs_hbm,
        idx_d, idx_s, rows, ones_v, acc_sh, deg_sh, sem):
    c = lax.axis_index("c")
    s = lax.axis_index("s")
    w = s * NUM_CORES + c

    for j in range(CHUNK // 16):
      ones_v[pl.ds(j * 16, 16)] = jnp.ones((16,), jnp.float32)

    row0 = s * ROWS_PER_TILE
    pltpu.sync_copy(z2_hbm.at[pl.ds(row0, ROWS_PER_TILE)],
                    acc_sh.at[pl.ds(row0, ROWS_PER_TILE)])

    @pl.when(s == 0)
    def _():
      pltpu.sync_copy(z2_hbm.at[pl.ds(ROWS_REM_OFF, ROWS_REM)],
                      acc_sh.at[pl.ds(ROWS_REM_OFF, ROWS_REM)])
      pltpu.sync_copy(z1_hbm, deg_sh)

    plsc.subcore_barrier()

    def body(kk, carry):
      chunk = w + kk * NW

      @pl.when(chunk < N_CHUNKS)
      def _():
        off = chunk * CHUNK
        pltpu.sync_copy(dst_hbm.at[pl.ds(off, CHUNK)], idx_d)
        pltpu.sync_copy(src_hbm.at[pl.ds(off, CHUNK)], idx_s)
        pltpu.async_copy(x_hbm.at[idx_d], rows, sem).wait()
        pltpu.sync_copy(rows, acc_sh.at[idx_s], add=True)
        pltpu.sync_copy(ones_v, deg_sh.at[idx_s], add=True)

      return carry

    lax.fori_loop(0, MAX_CHUNKS_PER_W, body, 0)

    plsc.subcore_barrier()

    pltpu.sync_copy(acc_sh.at[pl.ds(row0, ROWS_PER_TILE)],
                    parts_hbm.at[c, pl.ds(row0, ROWS_PER_TILE)])

    @pl.when(s == 0)
    def _():
      pltpu.sync_copy(acc_sh.at[pl.ds(ROWS_REM_OFF, ROWS_REM)],
                      parts_hbm.at[c, pl.ds(ROWS_REM_OFF, ROWS_REM)])
      pltpu.sync_copy(deg_sh, degs_hbm.at[c])

  return k(x, src, dst, zeros2d, zeros1d)


BLK = 2000


def _tc_combine(x, parts, degs_t, wst, bs, wnt, bn):
  def body(x_ref, p_ref, d_ref, ws_ref, bs_ref, wn_ref, bn_ref, o_ref):
    xb = x_ref[...]
    sm = jnp.dot(xb, ws_ref[...], preferred_element_type=jnp.float32)
    sm = sm + bs_ref[...]
    psum = p_ref[0] + p_ref[1]
    deg = d_ref[:, 0:1] + d_ref[:, 1:2]
    mean = psum / jnp.maximum(deg, 1.0)
    nm = jnp.dot(mean, wn_ref[...], preferred_element_type=jnp.float32)
    nm = jnp.where(deg > 0.0, nm + bn_ref[...], 0.0)
    o_ref[...] = jnp.maximum(sm + nm, 0.0)

  return pl.pallas_call(
      body,
      grid=(N_NODES // BLK,),
      in_specs=[
          pl.BlockSpec((BLK, DIM), lambda i: (i, 0)),
          pl.BlockSpec((NUM_CORES, BLK, DIM), lambda i: (0, i, 0)),
          pl.BlockSpec((BLK, NUM_CORES), lambda i: (i, 0)),
          pl.BlockSpec((DIM, DIM), lambda i: (0, 0)),
          pl.BlockSpec((1, DIM), lambda i: (0, 0)),
          pl.BlockSpec((DIM, DIM), lambda i: (0, 0)),
          pl.BlockSpec((1, DIM), lambda i: (0, 0)),
      ],
      out_specs=pl.BlockSpec((BLK, DIM), lambda i: (i, 0)),
      out_shape=jax.ShapeDtypeStruct((N_NODES, DIM), jnp.float32),
  )(x, parts, degs_t, wst, bs, wnt, bn)


def kernel(x, edge_index, W_self, b_self, W_neigh, b_neigh):
  src = edge_index[0]
  dst = edge_index[1]
  zeros2d = jnp.zeros((N_NODES, DIM), jnp.float32)
  zeros1d = jnp.zeros((N_NODES,), jnp.float32)
  parts, degs = _sc_segment_sum(x, src, dst, zeros2d, zeros1d)
  return _tc_combine(x, parts, degs.T, W_self.T, b_self[None, :],
                     W_neigh.T, b_neigh[None, :])
